# Initial kernel scaffold; baseline (speedup 1.0000x reference)
import jax
import jax.numpy as jnp
from jax.experimental import pallas as pl


def kernel(batch_pred, batch_truth, batch_label):
    raise NotImplementedError("write your pallas kernel here")


# speedup vs baseline: 4.9804x; 4.9804x over previous
import functools

import jax
import jax.numpy as jnp
from jax import lax
from jax.experimental import pallas as pl
from jax.experimental.pallas import tpu as pltpu

C = 1000
CP = 1024
D = 256
B = 16384
BLK = 2048
NB = B // BLK


def _accum_body(label_ref, pred_ref, truth_ref, sum_ref, cnt_ref):
    i = pl.program_id(0)
    pred = pred_ref[...]
    truth = truth_ref[...]
    labels = label_ref[0, 0, :]

    pn = jnp.sqrt(jnp.sum(pred * pred, axis=1, keepdims=True))
    p_ = pred / (pn + 1e-10)
    tn = jnp.sqrt(jnp.sum(truth * truth, axis=1, keepdims=True))
    t_ = truth / (tn + 1e-10)
    off = (p_ - t_) ** 2

    onehot = (labels[:, None] ==
              lax.broadcasted_iota(jnp.int32, (BLK, CP), 1)).astype(jnp.float32)
    part = lax.dot_general(onehot, off, (((0,), (0,)), ((), ())),
                           preferred_element_type=jnp.float32)
    ones = jnp.ones((BLK, 128), dtype=jnp.float32)
    cnt_part = lax.dot_general(onehot, ones, (((0,), (0,)), ((), ())),
                               preferred_element_type=jnp.float32)

    @pl.when(i == 0)
    def _init():
        sum_ref[...] = part
        cnt_ref[...] = cnt_part

    @pl.when(i > 0)
    def _acc():
        sum_ref[...] += part
        cnt_ref[...] += cnt_part


def _weights_body(sum_ref, cnt_ref, mean_ref, w_ref):
    s = sum_ref[...]
    cnt = cnt_ref[:, 0:1]
    mean = s / jnp.maximum(cnt, 1.0)
    mask = mean > 0.0
    big = jnp.where(mask, mean, jnp.inf)
    col_min = jnp.min(big, axis=1, keepdims=True)
    row_min = jnp.min(big, axis=0, keepdims=True)
    col_min = jnp.where(col_min < jnp.inf, col_min, 1.0)
    row_min = jnp.where(row_min < jnp.inf, row_min, 1.0)
    safe = jnp.where(mask, mean, 1.0)
    w1 = jnp.log(safe / row_min) + 1.0
    w2 = jnp.log(safe / col_min) + 1.0
    w = jnp.where(mask, w1 * w2, 1.0)
    mean_ref[...] = mean
    w_ref[...] = w


@jax.jit
def kernel(batch_pred, batch_truth, batch_label):
    labels3 = batch_label.reshape(NB, 1, BLK)
    sums, cnts = pl.pallas_call(
        _accum_body,
        grid=(NB,),
        in_specs=[
            pl.BlockSpec((1, 1, BLK), lambda i: (i, 0, 0)),
            pl.BlockSpec((BLK, D), lambda i: (i, 0)),
            pl.BlockSpec((BLK, D), lambda i: (i, 0)),
        ],
        out_specs=[
            pl.BlockSpec((CP, D), lambda i: (0, 0)),
            pl.BlockSpec((CP, 128), lambda i: (0, 0)),
        ],
        out_shape=[
            jax.ShapeDtypeStruct((CP, D), jnp.float32),
            jax.ShapeDtypeStruct((CP, 128), jnp.float32),
        ],
    )(labels3, batch_pred, batch_truth)

    mean_p, w_p = pl.pallas_call(
        _weights_body,
        out_shape=[
            jax.ShapeDtypeStruct((CP, D), jnp.float32),
            jax.ShapeDtypeStruct((CP, D), jnp.float32),
        ],
    )(sums, cnts)
    return (mean_p[:C], w_p[:C])
